# Initial kernel scaffold; baseline (speedup 1.0000x reference)
import jax
import jax.numpy as jnp
from jax.experimental import pallas as pl


def kernel(edge_rep, cycle_rep, edge_nodes, cycle5_nodes, cycle6_nodes, eW1, eg1, eb1, eW2, eg2, eb2, cW1, cg1, cb1, cW2, cg2, cb2):
    raise NotImplementedError("write your pallas kernel here")


# speedup vs baseline: 2.1042x; 2.1042x over previous
import jax
import jax.numpy as jnp
from jax import lax
from jax.experimental import pallas as pl
from jax.experimental.pallas import tpu as pltpu
from jax.experimental.pallas import tpu_sc as plsc

H = 128
NN = 10000
E = 160000
AE = 2 * E
C5 = 10000
C6 = 10000
A5 = 5 * C5
A6 = 6 * C6
AC = A5 + A6
NNP = NN + 8
NC, NS = 2, 16
NW = NC * NS
B5 = 57344
B6 = 65536
BEG = 327680
F32 = jnp.float32


def _sc_scatter(src, idx, init, *, btot, chunk, src_base=0):
    share = btot // NW
    nit = share // chunk
    mesh = plsc.VectorSubcoreMesh(core_axis_name="c", subcore_axis_name="s")

    def body(src_ref, idx_ref, init_ref, out_ref, idx_v, rows_v, table):
        c = lax.axis_index("c")
        s = lax.axis_index("s")
        wid = s * NC + c

        @pl.when(s == 0)
        def _():
            pltpu.sync_copy(init_ref.at[c], table)

        plsc.subcore_barrier()

        def step(j, carry):
            off = wid * share + j * chunk
            pltpu.sync_copy(idx_ref.at[pl.ds(off, chunk)], idx_v)
            pltpu.sync_copy(src_ref.at[pl.ds(src_base + off, chunk)], rows_v)
            pltpu.sync_copy(rows_v, table.at[idx_v], add=True)
            return carry

        lax.fori_loop(0, nit, step, 0)
        plsc.subcore_barrier()

        @pl.when(s == 0)
        def _():
            pltpu.sync_copy(table, out_ref.at[c])

    f = pl.kernel(
        body,
        out_type=jax.ShapeDtypeStruct((NC, NNP, H), F32),
        mesh=mesh,
        scratch_types=[
            pltpu.VMEM((chunk,), jnp.int32),
            pltpu.VMEM((chunk, H), F32),
            pltpu.VMEM_SHARED((NNP, H), F32),
        ],
    )
    return f(src, idx, init)


def _sc_gather(table, idx, *, btot, chunk, width):
    share = btot // NW
    nit = share // chunk
    mesh = plsc.VectorSubcoreMesh(core_axis_name="c", subcore_axis_name="s")

    def body(table_ref, idx_ref, out_ref, idx_v, rows_v, sem):
        c = lax.axis_index("c")
        s = lax.axis_index("s")
        wid = s * NC + c

        def step(j, carry):
            off = wid * share + j * chunk
            pltpu.sync_copy(idx_ref.at[pl.ds(off, chunk)], idx_v)
            pltpu.async_copy(table_ref.at[idx_v], rows_v, sem).wait()
            pltpu.sync_copy(rows_v, out_ref.at[pl.ds(off, chunk)])
            return carry

        lax.fori_loop(0, nit, step, 0)

    f = pl.kernel(
        body,
        out_type=jax.ShapeDtypeStruct((btot, width), F32),
        mesh=mesh,
        scratch_types=[
            pltpu.VMEM((chunk,), jnp.int32),
            pltpu.VMEM((chunk, width), F32),
            pltpu.SemaphoreType.DMA,
        ],
    )
    return f(table, idx)


_T_NP = 72


def _tc_add2(parts):
    def body(p_ref, o_ref):
        o_ref[...] = p_ref[0] + p_ref[1]

    return pl.pallas_call(
        body,
        grid=(NNP // _T_NP,),
        in_specs=[pl.BlockSpec((2, _T_NP, H), lambda i: (0, i, 0))],
        out_specs=pl.BlockSpec((_T_NP, H), lambda i: (i, 0)),
        out_shape=jax.ShapeDtypeStruct((NNP, H), F32),
    )(parts)


def _tc_gsum_rep(x, *, g, rows, tile, out_rows):
    def body(x_ref, o_ref):
        ri = lax.broadcasted_iota(jnp.int32, (tile, tile), 0) // g
        ci = lax.broadcasted_iota(jnp.int32, (tile, tile), 1) // g
        bmat = (ri == ci).astype(F32)
        o_ref[...] = jnp.dot(bmat, x_ref[...], preferred_element_type=F32)

    return pl.pallas_call(
        body,
        grid=(rows // tile,),
        in_specs=[pl.BlockSpec((tile, H), lambda i: (i, 0))],
        out_specs=pl.BlockSpec((tile, H), lambda i: (i, 0)),
        out_shape=jax.ShapeDtypeStruct((out_rows, H), F32),
    )(x)


def _tc_te(s2a, s2b, spa, spb, sr, wb, wc):
    def body(a_ref, b_ref, c_ref, d_ref, e_ref, wb_ref, wc_ref, t1_ref, tq_ref):
        t1 = jnp.zeros((_T_NP, 2 * H), F32)
        tq = jnp.zeros((_T_NP, 2 * H), F32)
        for k, r in enumerate((a_ref, b_ref, c_ref, d_ref, e_ref)):
            piece = r[0] + r[1]
            wbk = wb_ref[k * H:(k + 1) * H, :]
            wck = wc_ref[k * H:(k + 1) * H, :]
            t1 = t1 + jnp.dot(piece, wbk + wck, preferred_element_type=F32)
            tq = tq + jnp.dot(piece, wbk, preferred_element_type=F32)
        t1_ref[...] = t1
        tq_ref[...] = tq

    part = pl.BlockSpec((2, _T_NP, H), lambda i: (0, i, 0))
    wspec = pl.BlockSpec((5 * H, 2 * H), lambda i: (0, 0))
    return pl.pallas_call(
        body,
        grid=(NNP // _T_NP,),
        in_specs=[part, part, part, part, part, wspec, wspec],
        out_specs=[pl.BlockSpec((_T_NP, 2 * H), lambda i: (i, 0))] * 2,
        out_shape=[jax.ShapeDtypeStruct((NNP, 2 * H), F32)] * 2,
    )(s2a, s2b, spa, spb, sr, wb, wc)


def _tc_edge_pass1(er, g1, g2s, wa):
    tile = 640

    def body(er_ref, g1_ref, g2_ref, wa_ref, y_ref, st_ref):
        i = pl.program_id(0)
        y = (jnp.dot(er_ref[...], wa_ref[...], preferred_element_type=F32)
             + g1_ref[...] + g2_ref[...])
        y_ref[...] = y

        @pl.when(i == 0)
        def _():
            st_ref[...] = jnp.zeros_like(st_ref)

        st_ref[0:1, :] += jnp.sum(y, axis=0, keepdims=True)
        st_ref[1:2, :] += jnp.sum(y * y, axis=0, keepdims=True)

    return pl.pallas_call(
        body,
        grid=(AE // tile,),
        in_specs=[
            pl.BlockSpec((tile, H), lambda i: (i, 0)),
            pl.BlockSpec((tile, 2 * H), lambda i: (i, 0)),
            pl.BlockSpec((tile, 2 * H), lambda i: (i, 0)),
            pl.BlockSpec((H, 2 * H), lambda i: (0, 0)),
        ],
        out_specs=[
            pl.BlockSpec((tile, 2 * H), lambda i: (i, 0)),
            pl.BlockSpec((8, 2 * H), lambda i: (0, 0)),
        ],
        out_shape=[
            jax.ShapeDtypeStruct((AE, 2 * H), F32),
            jax.ShapeDtypeStruct((8, 2 * H), F32),
        ],
    )(er, g1, g2s, wa)


def _tc_cycle_pass1(p2a, p2b, r2a, r2b, crep, vca, vcb, vba, vbb, vr, *,
                    rows, row_off):
    tile = 400

    def body(p2a_ref, p2b_ref, r2a_ref, r2b_ref, cr_ref,
             vca_ref, vcb_ref, vba_ref, vbb_ref, vr_ref, y_ref, st_ref):
        i = pl.program_id(0)
        y = (jnp.dot(p2a_ref[...], vca_ref[...], preferred_element_type=F32)
             + jnp.dot(p2b_ref[...], vcb_ref[...], preferred_element_type=F32)
             + jnp.dot(r2a_ref[...], vba_ref[...], preferred_element_type=F32)
             + jnp.dot(r2b_ref[...], vbb_ref[...], preferred_element_type=F32)
             + jnp.dot(cr_ref[...], vr_ref[...], preferred_element_type=F32))
        y_ref[...] = y

        @pl.when(i == 0)
        def _():
            st_ref[...] = jnp.zeros_like(st_ref)

        st_ref[0:1, :] += jnp.sum(y, axis=0, keepdims=True)
        st_ref[1:2, :] += jnp.sum(y * y, axis=0, keepdims=True)

    row = pl.BlockSpec((tile, H), lambda i: (i, 0))
    woff = pl.BlockSpec((tile, H), lambda i: (i + row_off, 0))
    wmat = pl.BlockSpec((H, 2 * H), lambda i: (0, 0))
    return pl.pallas_call(
        body,
        grid=(rows // tile,),
        in_specs=[row, row, row, row, woff, wmat, wmat, wmat, wmat, wmat],
        out_specs=[
            pl.BlockSpec((tile, 2 * H), lambda i: (i, 0)),
            pl.BlockSpec((8, 2 * H), lambda i: (0, 0)),
        ],
        out_shape=[
            jax.ShapeDtypeStruct((rows, 2 * H), F32),
            jax.ShapeDtypeStruct((8, 2 * H), F32),
        ],
    )(p2a, p2b, r2a, r2b, crep, vca, vcb, vba, vbb, vr)


def _tc_pass2(y, st, w2, gg, bb, *, n_total, tile, rows):
    inv_n = 1.0 / float(n_total)

    def body(y_ref, st_ref, w2_ref, g_ref, b_ref, z_ref, st2_ref):
        i = pl.program_id(0)
        m = st_ref[0:1, :] * inv_n
        v = st_ref[1:2, :] * inv_n - m * m
        r = lax.rsqrt(v + 1e-5)
        h = jnp.maximum((y_ref[...] - m) * r * g_ref[...] + b_ref[...], 0.0)
        z = jnp.dot(h, w2_ref[...], preferred_element_type=F32)
        z_ref[...] = z

        @pl.when(i == 0)
        def _():
            st2_ref[...] = jnp.zeros_like(st2_ref)

        st2_ref[0:1, :] += jnp.sum(z, axis=0, keepdims=True)
        st2_ref[1:2, :] += jnp.sum(z * z, axis=0, keepdims=True)

    return pl.pallas_call(
        body,
        grid=(rows // tile,),
        in_specs=[
            pl.BlockSpec((tile, 2 * H), lambda i: (i, 0)),
            pl.BlockSpec((8, 2 * H), lambda i: (0, 0)),
            pl.BlockSpec((2 * H, H), lambda i: (0, 0)),
            pl.BlockSpec((1, 2 * H), lambda i: (0, 0)),
            pl.BlockSpec((1, 2 * H), lambda i: (0, 0)),
        ],
        out_specs=[
            pl.BlockSpec((tile, H), lambda i: (i, 0)),
            pl.BlockSpec((8, H), lambda i: (0, 0)),
        ],
        out_shape=[
            jax.ShapeDtypeStruct((rows, H), F32),
            jax.ShapeDtypeStruct((8, H), F32),
        ],
    )(y, st, w2, gg, bb)


def _tc_pass3(z, st2, gg, bb, *, n_total, tile, rows):
    inv_n = 1.0 / float(n_total)

    def body(z_ref, st_ref, g_ref, b_ref, o_ref):
        m = st_ref[0:1, :] * inv_n
        v = st_ref[1:2, :] * inv_n - m * m
        r = lax.rsqrt(v + 1e-5)
        o_ref[...] = jnp.maximum(
            (z_ref[...] - m) * r * g_ref[...] + b_ref[...], 0.0)

    return pl.pallas_call(
        body,
        grid=(rows // tile,),
        in_specs=[
            pl.BlockSpec((tile, H), lambda i: (i, 0)),
            pl.BlockSpec((8, H), lambda i: (0, 0)),
            pl.BlockSpec((1, H), lambda i: (0, 0)),
            pl.BlockSpec((1, H), lambda i: (0, 0)),
        ],
        out_specs=pl.BlockSpec((tile, H), lambda i: (i, 0)),
        out_shape=jax.ShapeDtypeStruct((rows, H), F32),
    )(z, st2, gg, bb)


def kernel(edge_rep, cycle_rep, edge_nodes, cycle5_nodes, cycle6_nodes,
           eW1, eg1, eb1, eW2, eg2, eb2, cW1, cg1, cb1, cW2, cg2, cb2):
    en = edge_nodes.astype(jnp.int32)
    c5 = cycle5_nodes.astype(jnp.int32)
    c6 = cycle6_nodes.astype(jnp.int32)

    padv = jnp.int32(NN)
    c5p = jnp.concatenate([c5, jnp.full((B5 - A5,), padv)])
    c6p = jnp.concatenate([c6, jnp.full((B6 - A6,), padv)])
    enp = jnp.concatenate([en, jnp.full((BEG - AE,), padv)])
    ens = en.reshape(E, 2)[:, ::-1].reshape(AE)
    ensp = jnp.concatenate([ens, jnp.full((BEG - AE,), padv)])
    c6shift = jnp.concatenate([jnp.full((B6 - A6,), padv), c6])

    zinit = jnp.zeros((NC, NNP, H), F32)

    ns_e = _tc_add2(_sc_scatter(edge_rep, en, zinit, btot=AE, chunk=200))
    g5 = _sc_gather(ns_e, c5p, btot=B5, chunk=256, width=H)
    g6 = _sc_gather(ns_e, c6p, btot=B6, chunk=256, width=H)
    r1_5 = _tc_gsum_rep(g5, g=5, rows=A5, tile=400, out_rows=B5)
    r1_6 = _tc_gsum_rep(g6, g=6, rows=A6, tile=480, out_rows=B6)

    t5a = _tc_add2(_sc_scatter(r1_5, c5p, zinit, btot=B5, chunk=256))
    t5b = _tc_add2(_sc_scatter(g5, c5p, zinit, btot=B5, chunk=256))
    t6a = _tc_add2(_sc_scatter(r1_6, c6p, zinit, btot=B6, chunk=256))
    t6b = _tc_add2(_sc_scatter(g6, c6p, zinit, btot=B6, chunk=256))

    p2_5a = _sc_gather(t5a, c5p, btot=B5, chunk=256, width=H)
    p2_5b = _sc_gather(t5b, c5p, btot=B5, chunk=256, width=H)
    p2_6a = _sc_gather(t6a, c6p, btot=B6, chunk=256, width=H)
    p2_6b = _sc_gather(t6b, c6p, btot=B6, chunk=256, width=H)

    r2_5a = _tc_gsum_rep(p2_5a, g=5, rows=A5, tile=400, out_rows=B5)
    r2_5b = _tc_gsum_rep(p2_5b, g=5, rows=A5, tile=400, out_rows=B5)
    r2_6a = _tc_gsum_rep(p2_6a, g=6, rows=A6, tile=480, out_rows=B6)
    r2_6b = _tc_gsum_rep(p2_6b, g=6, rows=A6, tile=480, out_rows=B6)

    s2a = _sc_scatter(r2_6a, c6p, _sc_scatter(r2_5a, c5p, zinit,
                                              btot=B5, chunk=256),
                      btot=B6, chunk=256)
    s2b = _sc_scatter(r2_6b, c6p, _sc_scatter(r2_5b, c5p, zinit,
                                              btot=B5, chunk=256),
                      btot=B6, chunk=256)
    spa = _sc_scatter(p2_6a, c6p, _sc_scatter(p2_5a, c5p, zinit,
                                              btot=B5, chunk=256),
                      btot=B6, chunk=256)
    spb = _sc_scatter(p2_6b, c6p, _sc_scatter(p2_5b, c5p, zinit,
                                              btot=B5, chunk=256),
                      btot=B6, chunk=256)
    sr = _sc_scatter(cycle_rep, c6shift,
                     _sc_scatter(cycle_rep, c5p, zinit, btot=B5, chunk=256),
                     btot=B6, chunk=256, src_base=AC - B6)

    wb = eW1[H:6 * H, :]
    wc = eW1[6 * H:, :]
    tg1, tq = _tc_te(s2a, s2b, spa, spb, sr, wb, wc)

    g1 = _sc_gather(tg1, enp, btot=BEG, chunk=256, width=2 * H)
    g2s = _sc_gather(tq, ensp, btot=BEG, chunk=256, width=2 * H)
    y_e, st_e = _tc_edge_pass1(edge_rep, g1, g2s, eW1[:H, :])
    z_e, st2_e = _tc_pass2(y_e, st_e, eW2, eg1.reshape(1, -1),
                           eb1.reshape(1, -1), n_total=AE, tile=640, rows=AE)
    edge_out = _tc_pass3(z_e, st2_e, eg2.reshape(1, -1), eb2.reshape(1, -1),
                         n_total=AE, tile=640, rows=AE)

    vb_a = cW1[0:H, :]
    vb_b = cW1[H:2 * H, :]
    vc_a = cW1[2 * H:3 * H, :]
    vc_b = cW1[3 * H:4 * H, :]
    vr = cW1[4 * H:, :]
    y5, st5 = _tc_cycle_pass1(p2_5a, p2_5b, r2_5a, r2_5b, cycle_rep,
                              vc_a, vc_b, vb_a, vb_b, vr,
                              rows=A5, row_off=0)
    y6, st6 = _tc_cycle_pass1(p2_6a, p2_6b, r2_6a, r2_6b, cycle_rep,
                              vc_a, vc_b, vb_a, vb_b, vr,
                              rows=A6, row_off=A5 // 400)
    st_c = st5 + st6
    cg1r = cg1.reshape(1, -1)
    cb1r = cb1.reshape(1, -1)
    z5, st2_5 = _tc_pass2(y5, st_c, cW2, cg1r, cb1r,
                          n_total=AC, tile=400, rows=A5)
    z6, st2_6 = _tc_pass2(y6, st_c, cW2, cg1r, cb1r,
                          n_total=AC, tile=400, rows=A6)
    st2_c = st2_5 + st2_6
    cg2r = cg2.reshape(1, -1)
    cb2r = cb2.reshape(1, -1)
    co5 = _tc_pass3(z5, st2_c, cg2r, cb2r, n_total=AC, tile=400, rows=A5)
    co6 = _tc_pass3(z6, st2_c, cg2r, cb2r, n_total=AC, tile=400, rows=A6)
    cycle_out = jnp.concatenate([co5, co6], axis=0)

    return edge_out, cycle_out


# speedup vs baseline: 2.1715x; 1.0320x over previous
import jax
import jax.numpy as jnp
from jax import lax
from jax.experimental import pallas as pl
from jax.experimental.pallas import tpu as pltpu
from jax.experimental.pallas import tpu_sc as plsc

H = 128
NN = 10000
E = 160000
AE = 2 * E
C5 = 10000
C6 = 10000
A5 = 5 * C5
A6 = 6 * C6
AC = A5 + A6
NNP = NN + 8
NC, NS = 2, 16
NW = NC * NS
B5 = 57344
B6 = 65536
BEG = 327680
F32 = jnp.float32


def _sc_scatter(src, idx, init, *, btot, chunk, src_base=0):
    share = btot // NW
    nit = share // chunk
    assert share % chunk == 0 and chunk % 8 == 0 and nit >= 2
    mesh = plsc.VectorSubcoreMesh(core_axis_name="c", subcore_axis_name="s")

    def body(src_ref, idx_ref, init_ref, out_ref, i0, i1, r0, r1, table,
             l0, l1, s0, s1):
        c = lax.axis_index("c")
        s = lax.axis_index("s")
        wid = s * NC + c
        base = wid * share
        idx_v = (i0, i1)
        rows_v = (r0, r1)
        lsem = (l0, l1)
        ssem = (s0, s1)

        @pl.when(s == 0)
        def _():
            pltpu.sync_copy(init_ref.at[c], table)

        plsc.subcore_barrier()

        def step(jj, carry):
            for b in range(2):
                off = base + (2 * jj + b) * chunk

                @pl.when(jj > 0)
                def _():
                    pltpu.make_async_copy(
                        rows_v[b], table.at[idx_v[b]], ssem[b]).wait()

                pltpu.sync_copy(idx_ref.at[pl.ds(off, chunk)], idx_v[b])
                pltpu.async_copy(src_ref.at[pl.ds(src_base + off, chunk)],
                                 rows_v[b], lsem[b])
            for b in range(2):
                pltpu.make_async_copy(
                    src_ref.at[pl.ds(src_base + base, chunk)],
                    rows_v[b], lsem[b]).wait()
                pltpu.async_copy(rows_v[b], table.at[idx_v[b]],
                                 ssem[b], add=True)
            return carry

        lax.fori_loop(0, nit // 2, step, 0)
        if nit % 2 == 1:
            off = base + (nit - 1) * chunk
            pltpu.make_async_copy(
                rows_v[0], table.at[idx_v[0]], ssem[0]).wait()
            pltpu.sync_copy(idx_ref.at[pl.ds(off, chunk)], idx_v[0])
            pltpu.sync_copy(src_ref.at[pl.ds(src_base + off, chunk)],
                            rows_v[0])
            pltpu.async_copy(rows_v[0], table.at[idx_v[0]], ssem[0],
                             add=True)
        for b in range(2):
            pltpu.make_async_copy(
                rows_v[b], table.at[idx_v[b]], ssem[b]).wait()
        plsc.subcore_barrier()

        @pl.when(s == 0)
        def _():
            pltpu.sync_copy(table, out_ref.at[c])

    f = pl.kernel(
        body,
        out_type=jax.ShapeDtypeStruct((NC, NNP, H), F32),
        mesh=mesh,
        scratch_types=[
            pltpu.VMEM((chunk,), jnp.int32),
            pltpu.VMEM((chunk,), jnp.int32),
            pltpu.VMEM((chunk, H), F32),
            pltpu.VMEM((chunk, H), F32),
            pltpu.VMEM_SHARED((NNP, H), F32),
            pltpu.SemaphoreType.DMA,
            pltpu.SemaphoreType.DMA,
            pltpu.SemaphoreType.DMA,
            pltpu.SemaphoreType.DMA,
        ],
    )
    return f(src, idx, init)


def _sc_gather(table, idx, *, btot, chunk, width):
    share = btot // NW
    nit = share // chunk
    assert share % chunk == 0 and chunk % 8 == 0 and nit >= 2
    mesh = plsc.VectorSubcoreMesh(core_axis_name="c", subcore_axis_name="s")

    def body(table_ref, idx_ref, out_ref, i0, i1, r0, r1, g0, g1, s0, s1):
        c = lax.axis_index("c")
        s = lax.axis_index("s")
        wid = s * NC + c
        base = wid * share
        idx_v = (i0, i1)
        rows_v = (r0, r1)
        gsem = (g0, g1)
        ssem = (s0, s1)

        def step(jj, carry):
            for b in range(2):
                off = base + (2 * jj + b) * chunk

                @pl.when(jj > 0)
                def _():
                    pltpu.make_async_copy(
                        rows_v[b], out_ref.at[pl.ds(base, chunk)],
                        ssem[b]).wait()

                pltpu.sync_copy(idx_ref.at[pl.ds(off, chunk)], idx_v[b])
                pltpu.async_copy(table_ref.at[idx_v[b]], rows_v[b],
                                 gsem[b])
            for b in range(2):
                off = base + (2 * jj + b) * chunk
                pltpu.make_async_copy(table_ref.at[idx_v[b]],
                                      rows_v[b], gsem[b]).wait()
                pltpu.async_copy(rows_v[b], out_ref.at[pl.ds(off, chunk)],
                                 ssem[b])
            return carry

        lax.fori_loop(0, nit // 2, step, 0)
        if nit % 2 == 1:
            off = base + (nit - 1) * chunk
            pltpu.make_async_copy(
                rows_v[0], out_ref.at[pl.ds(base, chunk)], ssem[0]).wait()
            pltpu.sync_copy(idx_ref.at[pl.ds(off, chunk)], idx_v[0])
            pltpu.async_copy(table_ref.at[idx_v[0]], rows_v[0], gsem[0])
            pltpu.make_async_copy(table_ref.at[idx_v[0]], rows_v[0],
                                  gsem[0]).wait()
            pltpu.async_copy(rows_v[0], out_ref.at[pl.ds(off, chunk)],
                             ssem[0])
        for b in range(2):
            pltpu.make_async_copy(
                rows_v[b], out_ref.at[pl.ds(base, chunk)], ssem[b]).wait()

    f = pl.kernel(
        body,
        out_type=jax.ShapeDtypeStruct((btot, width), F32),
        mesh=mesh,
        scratch_types=[
            pltpu.VMEM((chunk,), jnp.int32),
            pltpu.VMEM((chunk,), jnp.int32),
            pltpu.VMEM((chunk, width), F32),
            pltpu.VMEM((chunk, width), F32),
            pltpu.SemaphoreType.DMA,
            pltpu.SemaphoreType.DMA,
            pltpu.SemaphoreType.DMA,
            pltpu.SemaphoreType.DMA,
        ],
    )
    return f(table, idx)


_T_NP = 72


def _tc_add2(parts):
    def body(p_ref, o_ref):
        o_ref[...] = p_ref[0] + p_ref[1]

    return pl.pallas_call(
        body,
        grid=(NNP // _T_NP,),
        in_specs=[pl.BlockSpec((2, _T_NP, H), lambda i: (0, i, 0))],
        out_specs=pl.BlockSpec((_T_NP, H), lambda i: (i, 0)),
        out_shape=jax.ShapeDtypeStruct((NNP, H), F32),
    )(parts)


def _tc_gsum_rep(x, *, g, rows, tile, out_rows):
    def body(x_ref, o_ref):
        ri = lax.broadcasted_iota(jnp.int32, (tile, tile), 0) // g
        ci = lax.broadcasted_iota(jnp.int32, (tile, tile), 1) // g
        bmat = (ri == ci).astype(F32)
        o_ref[...] = jnp.dot(bmat, x_ref[...], preferred_element_type=F32)

    return pl.pallas_call(
        body,
        grid=(rows // tile,),
        in_specs=[pl.BlockSpec((tile, H), lambda i: (i, 0))],
        out_specs=pl.BlockSpec((tile, H), lambda i: (i, 0)),
        out_shape=jax.ShapeDtypeStruct((out_rows, H), F32),
    )(x)


def _tc_te(s2a, s2b, spa, spb, sr, wb, wc):
    def body(a_ref, b_ref, c_ref, d_ref, e_ref, wb_ref, wc_ref, t1_ref, tq_ref):
        t1 = jnp.zeros((_T_NP, 2 * H), F32)
        tq = jnp.zeros((_T_NP, 2 * H), F32)
        for k, r in enumerate((a_ref, b_ref, c_ref, d_ref, e_ref)):
            piece = r[0] + r[1]
            wbk = wb_ref[k * H:(k + 1) * H, :]
            wck = wc_ref[k * H:(k + 1) * H, :]
            t1 = t1 + jnp.dot(piece, wbk + wck, preferred_element_type=F32)
            tq = tq + jnp.dot(piece, wbk, preferred_element_type=F32)
        t1_ref[...] = t1
        tq_ref[...] = tq

    part = pl.BlockSpec((2, _T_NP, H), lambda i: (0, i, 0))
    wspec = pl.BlockSpec((5 * H, 2 * H), lambda i: (0, 0))
    return pl.pallas_call(
        body,
        grid=(NNP // _T_NP,),
        in_specs=[part, part, part, part, part, wspec, wspec],
        out_specs=[pl.BlockSpec((_T_NP, 2 * H), lambda i: (i, 0))] * 2,
        out_shape=[jax.ShapeDtypeStruct((NNP, 2 * H), F32)] * 2,
    )(s2a, s2b, spa, spb, sr, wb, wc)


def _tc_edge_pass1(er, g1, g2s, wa):
    tile = 640

    def body(er_ref, g1_ref, g2_ref, wa_ref, y_ref, st_ref):
        i = pl.program_id(0)
        y = (jnp.dot(er_ref[...], wa_ref[...], preferred_element_type=F32)
             + g1_ref[...] + g2_ref[...])
        y_ref[...] = y

        @pl.when(i == 0)
        def _():
            st_ref[...] = jnp.zeros_like(st_ref)

        st_ref[0:1, :] += jnp.sum(y, axis=0, keepdims=True)
        st_ref[1:2, :] += jnp.sum(y * y, axis=0, keepdims=True)

    return pl.pallas_call(
        body,
        grid=(AE // tile,),
        in_specs=[
            pl.BlockSpec((tile, H), lambda i: (i, 0)),
            pl.BlockSpec((tile, 2 * H), lambda i: (i, 0)),
            pl.BlockSpec((tile, 2 * H), lambda i: (i, 0)),
            pl.BlockSpec((H, 2 * H), lambda i: (0, 0)),
        ],
        out_specs=[
            pl.BlockSpec((tile, 2 * H), lambda i: (i, 0)),
            pl.BlockSpec((8, 2 * H), lambda i: (0, 0)),
        ],
        out_shape=[
            jax.ShapeDtypeStruct((AE, 2 * H), F32),
            jax.ShapeDtypeStruct((8, 2 * H), F32),
        ],
    )(er, g1, g2s, wa)


def _tc_cycle_pass1(p2a, p2b, r2a, r2b, crep, vca, vcb, vba, vbb, vr, *,
                    rows, row_off):
    tile = 400

    def body(p2a_ref, p2b_ref, r2a_ref, r2b_ref, cr_ref,
             vca_ref, vcb_ref, vba_ref, vbb_ref, vr_ref, y_ref, st_ref):
        i = pl.program_id(0)
        y = (jnp.dot(p2a_ref[...], vca_ref[...], preferred_element_type=F32)
             + jnp.dot(p2b_ref[...], vcb_ref[...], preferred_element_type=F32)
             + jnp.dot(r2a_ref[...], vba_ref[...], preferred_element_type=F32)
             + jnp.dot(r2b_ref[...], vbb_ref[...], preferred_element_type=F32)
             + jnp.dot(cr_ref[...], vr_ref[...], preferred_element_type=F32))
        y_ref[...] = y

        @pl.when(i == 0)
        def _():
            st_ref[...] = jnp.zeros_like(st_ref)

        st_ref[0:1, :] += jnp.sum(y, axis=0, keepdims=True)
        st_ref[1:2, :] += jnp.sum(y * y, axis=0, keepdims=True)

    row = pl.BlockSpec((tile, H), lambda i: (i, 0))
    woff = pl.BlockSpec((tile, H), lambda i: (i + row_off, 0))
    wmat = pl.BlockSpec((H, 2 * H), lambda i: (0, 0))
    return pl.pallas_call(
        body,
        grid=(rows // tile,),
        in_specs=[row, row, row, row, woff, wmat, wmat, wmat, wmat, wmat],
        out_specs=[
            pl.BlockSpec((tile, 2 * H), lambda i: (i, 0)),
            pl.BlockSpec((8, 2 * H), lambda i: (0, 0)),
        ],
        out_shape=[
            jax.ShapeDtypeStruct((rows, 2 * H), F32),
            jax.ShapeDtypeStruct((8, 2 * H), F32),
        ],
    )(p2a, p2b, r2a, r2b, crep, vca, vcb, vba, vbb, vr)


def _tc_pass2(y, st, w2, gg, bb, *, n_total, tile, rows):
    inv_n = 1.0 / float(n_total)

    def body(y_ref, st_ref, w2_ref, g_ref, b_ref, z_ref, st2_ref):
        i = pl.program_id(0)
        m = st_ref[0:1, :] * inv_n
        v = st_ref[1:2, :] * inv_n - m * m
        r = lax.rsqrt(v + 1e-5)
        h = jnp.maximum((y_ref[...] - m) * r * g_ref[...] + b_ref[...], 0.0)
        z = jnp.dot(h, w2_ref[...], preferred_element_type=F32)
        z_ref[...] = z

        @pl.when(i == 0)
        def _():
            st2_ref[...] = jnp.zeros_like(st2_ref)

        st2_ref[0:1, :] += jnp.sum(z, axis=0, keepdims=True)
        st2_ref[1:2, :] += jnp.sum(z * z, axis=0, keepdims=True)

    return pl.pallas_call(
        body,
        grid=(rows // tile,),
        in_specs=[
            pl.BlockSpec((tile, 2 * H), lambda i: (i, 0)),
            pl.BlockSpec((8, 2 * H), lambda i: (0, 0)),
            pl.BlockSpec((2 * H, H), lambda i: (0, 0)),
            pl.BlockSpec((1, 2 * H), lambda i: (0, 0)),
            pl.BlockSpec((1, 2 * H), lambda i: (0, 0)),
        ],
        out_specs=[
            pl.BlockSpec((tile, H), lambda i: (i, 0)),
            pl.BlockSpec((8, H), lambda i: (0, 0)),
        ],
        out_shape=[
            jax.ShapeDtypeStruct((rows, H), F32),
            jax.ShapeDtypeStruct((8, H), F32),
        ],
    )(y, st, w2, gg, bb)


def _tc_pass3(z, st2, gg, bb, *, n_total, tile, rows):
    inv_n = 1.0 / float(n_total)

    def body(z_ref, st_ref, g_ref, b_ref, o_ref):
        m = st_ref[0:1, :] * inv_n
        v = st_ref[1:2, :] * inv_n - m * m
        r = lax.rsqrt(v + 1e-5)
        o_ref[...] = jnp.maximum(
            (z_ref[...] - m) * r * g_ref[...] + b_ref[...], 0.0)

    return pl.pallas_call(
        body,
        grid=(rows // tile,),
        in_specs=[
            pl.BlockSpec((tile, H), lambda i: (i, 0)),
            pl.BlockSpec((8, H), lambda i: (0, 0)),
            pl.BlockSpec((1, H), lambda i: (0, 0)),
            pl.BlockSpec((1, H), lambda i: (0, 0)),
        ],
        out_specs=pl.BlockSpec((tile, H), lambda i: (i, 0)),
        out_shape=jax.ShapeDtypeStruct((rows, H), F32),
    )(z, st2, gg, bb)


def kernel(edge_rep, cycle_rep, edge_nodes, cycle5_nodes, cycle6_nodes,
           eW1, eg1, eb1, eW2, eg2, eb2, cW1, cg1, cb1, cW2, cg2, cb2):
    en = edge_nodes.astype(jnp.int32)
    c5 = cycle5_nodes.astype(jnp.int32)
    c6 = cycle6_nodes.astype(jnp.int32)

    padv = jnp.int32(NN)
    c5p = jnp.concatenate([c5, jnp.full((B5 - A5,), padv)])
    c6p = jnp.concatenate([c6, jnp.full((B6 - A6,), padv)])
    enp = jnp.concatenate([en, jnp.full((BEG - AE,), padv)])
    ens = en.reshape(E, 2)[:, ::-1].reshape(AE)
    ensp = jnp.concatenate([ens, jnp.full((BEG - AE,), padv)])
    c6shift = jnp.concatenate([jnp.full((B6 - A6,), padv), c6])

    zinit = jnp.zeros((NC, NNP, H), F32)

    ns_e = _tc_add2(_sc_scatter(edge_rep, en, zinit, btot=AE, chunk=80))
    g5 = _sc_gather(ns_e, c5p, btot=B5, chunk=224, width=H)
    g6 = _sc_gather(ns_e, c6p, btot=B6, chunk=256, width=H)
    r1_5 = _tc_gsum_rep(g5, g=5, rows=A5, tile=400, out_rows=B5)
    r1_6 = _tc_gsum_rep(g6, g=6, rows=A6, tile=480, out_rows=B6)

    t5a = _tc_add2(_sc_scatter(r1_5, c5p, zinit, btot=B5, chunk=112))
    t5b = _tc_add2(_sc_scatter(g5, c5p, zinit, btot=B5, chunk=112))
    t6a = _tc_add2(_sc_scatter(r1_6, c6p, zinit, btot=B6, chunk=128))
    t6b = _tc_add2(_sc_scatter(g6, c6p, zinit, btot=B6, chunk=128))

    p2_5a = _sc_gather(t5a, c5p, btot=B5, chunk=224, width=H)
    p2_5b = _sc_gather(t5b, c5p, btot=B5, chunk=224, width=H)
    p2_6a = _sc_gather(t6a, c6p, btot=B6, chunk=256, width=H)
    p2_6b = _sc_gather(t6b, c6p, btot=B6, chunk=256, width=H)

    r2_5a = _tc_gsum_rep(p2_5a, g=5, rows=A5, tile=400, out_rows=B5)
    r2_5b = _tc_gsum_rep(p2_5b, g=5, rows=A5, tile=400, out_rows=B5)
    r2_6a = _tc_gsum_rep(p2_6a, g=6, rows=A6, tile=480, out_rows=B6)
    r2_6b = _tc_gsum_rep(p2_6b, g=6, rows=A6, tile=480, out_rows=B6)

    s2a = _sc_scatter(r2_6a, c6p, _sc_scatter(r2_5a, c5p, zinit,
                                              btot=B5, chunk=112),
                      btot=B6, chunk=128)
    s2b = _sc_scatter(r2_6b, c6p, _sc_scatter(r2_5b, c5p, zinit,
                                              btot=B5, chunk=112),
                      btot=B6, chunk=128)
    spa = _sc_scatter(p2_6a, c6p, _sc_scatter(p2_5a, c5p, zinit,
                                              btot=B5, chunk=112),
                      btot=B6, chunk=128)
    spb = _sc_scatter(p2_6b, c6p, _sc_scatter(p2_5b, c5p, zinit,
                                              btot=B5, chunk=112),
                      btot=B6, chunk=128)
    sr = _sc_scatter(cycle_rep, c6shift,
                     _sc_scatter(cycle_rep, c5p, zinit, btot=B5, chunk=112),
                     btot=B6, chunk=128, src_base=AC - B6)

    wb = eW1[H:6 * H, :]
    wc = eW1[6 * H:, :]
    tg1, tq = _tc_te(s2a, s2b, spa, spb, sr, wb, wc)

    g1 = _sc_gather(tg1, enp, btot=BEG, chunk=128, width=2 * H)
    g2s = _sc_gather(tq, ensp, btot=BEG, chunk=128, width=2 * H)
    y_e, st_e = _tc_edge_pass1(edge_rep, g1, g2s, eW1[:H, :])
    z_e, st2_e = _tc_pass2(y_e, st_e, eW2, eg1.reshape(1, -1),
                           eb1.reshape(1, -1), n_total=AE, tile=640, rows=AE)
    edge_out = _tc_pass3(z_e, st2_e, eg2.reshape(1, -1), eb2.reshape(1, -1),
                         n_total=AE, tile=640, rows=AE)

    vb_a = cW1[0:H, :]
    vb_b = cW1[H:2 * H, :]
    vc_a = cW1[2 * H:3 * H, :]
    vc_b = cW1[3 * H:4 * H, :]
    vr = cW1[4 * H:, :]
    y5, st5 = _tc_cycle_pass1(p2_5a, p2_5b, r2_5a, r2_5b, cycle_rep,
                              vc_a, vc_b, vb_a, vb_b, vr,
                              rows=A5, row_off=0)
    y6, st6 = _tc_cycle_pass1(p2_6a, p2_6b, r2_6a, r2_6b, cycle_rep,
                              vc_a, vc_b, vb_a, vb_b, vr,
                              rows=A6, row_off=A5 // 400)
    st_c = st5 + st6
    cg1r = cg1.reshape(1, -1)
    cb1r = cb1.reshape(1, -1)
    z5, st2_5 = _tc_pass2(y5, st_c, cW2, cg1r, cb1r,
                          n_total=AC, tile=400, rows=A5)
    z6, st2_6 = _tc_pass2(y6, st_c, cW2, cg1r, cb1r,
                          n_total=AC, tile=400, rows=A6)
    st2_c = st2_5 + st2_6
    cg2r = cg2.reshape(1, -1)
    cb2r = cb2.reshape(1, -1)
    co5 = _tc_pass3(z5, st2_c, cg2r, cb2r, n_total=AC, tile=400, rows=A5)
    co6 = _tc_pass3(z6, st2_c, cg2r, cb2r, n_total=AC, tile=400, rows=A6)
    cycle_out = jnp.concatenate([co5, co6], axis=0)

    return edge_out, cycle_out
